# baseline (speedup 1.0000x reference)
import functools

import jax
import jax.numpy as jnp
from jax import lax
from jax.experimental import pallas as pl
from jax.experimental.pallas import tpu as pltpu
from jax.experimental.pallas import tpu_sc as plsc

N_SETS = 16
N_POINTS = 4096
N_MARKS = 3
N_CLASSES = 32
H = W = 512
N_TOTAL = N_SETS * N_POINTS

NC = 2
NS = 16
NW = NC * NS
N_PER_W = N_TOTAL // NW
ROUND = 512
NROUND = N_PER_W // ROUND
LANES = 16
GROUPS = ROUND // LANES

_POS_ROWS = 4
_MARK_ROWS = 8
_N_ROWS = _POS_ROWS + N_MARKS * _MARK_ROWS

_SPATIAL_OFF = (0, 1, W, W + 1)
_PLANE = H * W
_WHALF = W // 2
_WPLANE = H * _WHALF


def _sc_body(py_hbm, px_hbm, c0_hbm, c1_hbm, c2_hbm, mask_hbm,
             pos_hbm, mm0_hbm, mm1_hbm, mm2_hbm, out_hbm, *scratch):
  pts_v = scratch[0:5]
  mask_v = scratch[5]
  p = 6
  idx_a = scratch[p:p + 4]; p += 4
  val_a = scratch[p:p + 4]; p += 4
  idx_b = scratch[p:p + 4]; p += 4
  val_b = scratch[p:p + 4]; p += 4
  out_v = scratch[p:p + 4]; p += 4
  sem_a, sem_b = scratch[p], scratch[p + 1]

  wid = lax.axis_index("s") * NC + lax.axis_index("c")
  base = wid * N_PER_W

  comp_hbm = (py_hbm, px_hbm, c0_hbm, c1_hbm, c2_hbm)
  for r in range(5):
    pltpu.sync_copy(comp_hbm[r].at[pl.ds(base, N_PER_W)], pts_v[r])
  pltpu.sync_copy(mask_hbm.at[pl.ds(base, N_PER_W)], mask_v)

  map_refs = (pos_hbm, mm0_hbm, mm1_hbm, mm2_hbm)

  def compute_idx(ci, idx):
    def gbody(g, _):
      off = ci * ROUND + g * LANES
      col = g * LANES
      ty = pts_v[0][pl.ds(off, LANES)] * float(H - 1)
      tx = pts_v[1][pl.ds(off, LANES)] * float(W - 1)
      y0 = jnp.minimum(ty.astype(jnp.int32), H - 2)
      x0 = jnp.minimum(tx.astype(jnp.int32), W - 2)
      s00 = y0 * W + x0
      for k in range(_POS_ROWS):
        idx[0][pl.ds(k * ROUND + col, LANES)] = s00 + _SPATIAL_OFF[k]
      xw0 = jnp.bitwise_and(x0, _WHALF - 1)
      xw1 = jnp.bitwise_and(x0 + 1, _WHALF - 1)
      row0 = y0 * _WHALF
      wcorner = (row0 + xw0, row0 + xw1,
                 row0 + _WHALF + xw0, row0 + _WHALF + xw1)
      for i in range(N_MARKS):
        c = pts_v[2 + i][pl.ds(off, LANES)] * float(N_CLASSES - 1)
        c0 = jnp.minimum(c.astype(jnp.int32), N_CLASSES - 2)
        b = c0 * _WPLANE
        for k in range(_MARK_ROWS):
          o = wcorner[k % 4]
          plane = _WPLANE if k >= 4 else 0
          idx[1 + i][pl.ds(k * ROUND + col, LANES)] = b + o + plane
      return 0

    lax.fori_loop(0, GROUPS, gbody, 0)

  _SUB = 1

  def _seg(m):
    rows = _POS_ROWS if m == 0 else _MARK_ROWS
    return rows * ROUND // _SUB

  def fire(idx, val, sem):
    for m in range(4):
      n = _seg(m)
      for s2 in range(_SUB):
        pltpu.async_copy(map_refs[m].at[idx[m].at[pl.ds(s2 * n, n)]],
                         val[m].at[pl.ds(s2 * n, n)], sem)

  def drain(idx, val, sem):
    for m in range(4):
      n = _seg(m)
      for s2 in range(_SUB):
        pltpu.make_async_copy(map_refs[m].at[idx[m].at[pl.ds(s2 * n, n)]],
                              val[m].at[pl.ds(s2 * n, n)], sem).wait()

  def interp(ci, val):
    def gbody(g, _):
      off = ci * ROUND + g * LANES
      col = g * LANES
      ty = pts_v[0][pl.ds(off, LANES)] * float(H - 1)
      tx = pts_v[1][pl.ds(off, LANES)] * float(W - 1)
      y0 = jnp.minimum(ty.astype(jnp.int32), H - 2)
      x0 = jnp.minimum(tx.astype(jnp.int32), W - 2)
      wy = ty - y0.astype(jnp.float32)
      wx = tx - x0.astype(jnp.float32)
      msk = mask_v[pl.ds(off, LANES)]

      def bilerp(v00, v01, v10, v11):
        top = v00 + wx * (v01 - v00)
        bot = v10 + wx * (v11 - v10)
        return top + wy * (bot - top)

      pvals = [val[0][pl.ds(k * ROUND + col, LANES)] for k in range(_POS_ROWS)]
      out_v[0][pl.ds(off, LANES)] = bilerp(*pvals) * msk
      hi0 = (x0 >> 8) == 1
      hi1 = ((x0 + 1) >> 8) == 1
      for i in range(N_MARKS):
        c = pts_v[2 + i][pl.ds(off, LANES)] * float(N_CLASSES - 1)
        c0 = jnp.minimum(c.astype(jnp.int32), N_CLASSES - 2)
        wc = c - c0.astype(jnp.float32)
        mvals = []
        for k in range(_MARK_ROWS):
          word = val[1 + i][pl.ds(k * ROUND + col, LANES)]
          u = plsc.bitcast(word, jnp.uint32)
          picked = jnp.where(hi1 if (k & 1) else hi0,
                             jnp.bitwise_and(u, jnp.uint32(0xFFFF0000)),
                             u << 16)
          mvals.append(plsc.bitcast(picked, jnp.float32))
        p0 = bilerp(*mvals[0:4])
        p1 = bilerp(*mvals[4:8])
        out_v[1 + i][pl.ds(off, LANES)] = (p0 + wc * (p1 - p0)) * msk
      return 0

    lax.fori_loop(0, GROUPS, gbody, 0)

  compute_idx(0, idx_a)
  fire(idx_a, val_a, sem_a)

  def pair_body(j, _):
    c0 = 2 * j
    compute_idx(c0 + 1, idx_b)
    fire(idx_b, val_b, sem_b)
    drain(idx_a, val_a, sem_a)
    interp(c0, val_a)
    compute_idx(c0 + 2, idx_a)
    fire(idx_a, val_a, sem_a)
    drain(idx_b, val_b, sem_b)
    interp(c0 + 1, val_b)
    return 0

  lax.fori_loop(0, NROUND // 2 - 1, pair_body, 0)

  compute_idx(NROUND - 1, idx_b)
  fire(idx_b, val_b, sem_b)
  drain(idx_a, val_a, sem_a)
  interp(NROUND - 2, val_a)
  drain(idx_b, val_b, sem_b)
  interp(NROUND - 1, val_b)

  for k in range(1 + N_MARKS):
    pltpu.sync_copy(out_v[k], out_hbm.at[pl.ds(k * N_TOTAL + base, N_PER_W)])


@jax.jit
def _sc_call(py, px, c0, c1, c2, mask, pos_map, mm0, mm1, mm2):
  mesh = plsc.VectorSubcoreMesh(core_axis_name="c", subcore_axis_name="s")
  return pl.kernel(
      _sc_body,
      out_type=jax.ShapeDtypeStruct(((1 + N_MARKS) * N_TOTAL,), jnp.float32),
      mesh=mesh,
      compiler_params=pltpu.CompilerParams(needs_layout_passes=False),
      scratch_types=(
          [pltpu.VMEM((N_PER_W,), jnp.float32) for _ in range(5)]
          + [pltpu.VMEM((N_PER_W,), jnp.float32)]
          + [pltpu.VMEM((_POS_ROWS * ROUND,), jnp.int32)]
          + [pltpu.VMEM((_MARK_ROWS * ROUND,), jnp.int32) for _ in range(3)]
          + [pltpu.VMEM((_POS_ROWS * ROUND,), jnp.float32)]
          + [pltpu.VMEM((_MARK_ROWS * ROUND,), jnp.float32) for _ in range(3)]
          + [pltpu.VMEM((_POS_ROWS * ROUND,), jnp.int32)]
          + [pltpu.VMEM((_MARK_ROWS * ROUND,), jnp.int32) for _ in range(3)]
          + [pltpu.VMEM((_POS_ROWS * ROUND,), jnp.float32)]
          + [pltpu.VMEM((_MARK_ROWS * ROUND,), jnp.float32) for _ in range(3)]
          + [pltpu.VMEM((N_PER_W,), jnp.float32) for _ in range(4)]
          + [pltpu.SemaphoreType.DMA, pltpu.SemaphoreType.DMA]
      ),
  )(py, px, c0, c1, c2, mask, pos_map, mm0, mm1, mm2)


def _pack_body(m_ref, out_ref):
  u = jax.lax.bitcast_convert_type(m_ref[0], jnp.uint32)
  rnd = (u + jnp.uint32(0x7FFF) + ((u >> 16) & jnp.uint32(1))) >> 16
  lo = rnd[:, :_WHALF]
  hi = rnd[:, _WHALF:]
  w = jnp.bitwise_or(lo, hi << 16)
  out_ref[...] = jax.lax.bitcast_convert_type(w, jnp.float32).reshape(16, 128)


@jax.jit
def _pack_map(m):
  packed = pl.pallas_call(
      _pack_body,
      grid=(N_CLASSES, H // 8),
      in_specs=[pl.BlockSpec((1, 8, W), lambda c, y: (c, y, 0))],
      out_specs=pl.BlockSpec((16, 128), lambda c, y: (c * (H // 8) + y, 0)),
      out_shape=jax.ShapeDtypeStruct((N_CLASSES * H * 2, 128), jnp.float32),
  )(m)
  return packed.reshape(N_CLASSES * _WPLANE)


def kernel(points, points_mask, position_energy_map,
           marks_energy_map_0, marks_energy_map_1, marks_energy_map_2):
  pts = points.reshape(N_TOTAL, 2 + N_MARKS)
  comps = [pts[:, r] for r in range(5)]
  mask = points_mask.reshape(N_TOTAL)
  pos_map = position_energy_map.reshape(H * W)
  mm0 = _pack_map(marks_energy_map_0)
  mm1 = _pack_map(marks_energy_map_1)
  mm2 = _pack_map(marks_energy_map_2)
  out = _sc_call(*comps, mask, pos_map, mm0, mm1, mm2)
  return out.reshape(1 + N_MARKS, N_SETS, N_POINTS)


# speedup vs baseline: 15.0584x; 15.0584x over previous
import functools

import jax
import jax.numpy as jnp
from jax import lax
from jax.experimental import pallas as pl
from jax.experimental.pallas import tpu as pltpu
from jax.experimental.pallas import tpu_sc as plsc

N_SETS = 16
N_POINTS = 4096
N_MARKS = 3
N_CLASSES = 32
H = W = 512
N_TOTAL = N_SETS * N_POINTS

NC = 2
NS = 16
NW = NC * NS
N_PER_W = N_TOTAL // NW
ROUND = 512
NROUND = N_PER_W // ROUND
LANES = 16
GROUPS = ROUND // LANES

_POS_ROWS = 4
_MARK_ROWS = 8
_N_ROWS = _POS_ROWS + N_MARKS * _MARK_ROWS

_SPATIAL_OFF = (0, 1, W, W + 1)
_PLANE = H * W
_WHALF = W // 2
_WPLANE = H * _WHALF


def _sc_body(py_hbm, px_hbm, c0_hbm, c1_hbm, c2_hbm, mask_hbm,
             pos_hbm, mm0_hbm, mm1_hbm, mm2_hbm, out_hbm, *scratch):
  pts_v = scratch[0:5]
  mask_v = scratch[5]
  p = 6
  idx_a = scratch[p:p + 4]; p += 4
  val_a = scratch[p:p + 4]; p += 4
  idx_b = scratch[p:p + 4]; p += 4
  val_b = scratch[p:p + 4]; p += 4
  out_v = scratch[p:p + 4]; p += 4
  sem_a, sem_b = scratch[p], scratch[p + 1]

  wid = lax.axis_index("s") * NC + lax.axis_index("c")
  base = wid * N_PER_W

  comp_hbm = (py_hbm, px_hbm, c0_hbm, c1_hbm, c2_hbm)
  for r in range(5):
    pltpu.sync_copy(comp_hbm[r].at[pl.ds(base, N_PER_W)], pts_v[r])
  pltpu.sync_copy(mask_hbm.at[pl.ds(base, N_PER_W)], mask_v)

  map_refs = (pos_hbm, mm0_hbm, mm1_hbm, mm2_hbm)

  def compute_idx(ci, idx):
    def gbody(g, _):
      off = ci * ROUND + g * LANES
      col = g * LANES
      ty = pts_v[0][pl.ds(off, LANES)] * float(H - 1)
      tx = pts_v[1][pl.ds(off, LANES)] * float(W - 1)
      y0 = jnp.minimum(ty.astype(jnp.int32), H - 2)
      x0 = jnp.minimum(tx.astype(jnp.int32), W - 2)
      s00 = y0 * W + x0
      for k in range(_POS_ROWS):
        idx[0][pl.ds(k * ROUND + col, LANES)] = s00 + _SPATIAL_OFF[k]
      xw0 = jnp.bitwise_and(x0, _WHALF - 1)
      xw1 = jnp.bitwise_and(x0 + 1, _WHALF - 1)
      row0 = y0 * _WHALF
      wcorner = (row0 + xw0, row0 + xw1,
                 row0 + _WHALF + xw0, row0 + _WHALF + xw1)
      for i in range(N_MARKS):
        c = pts_v[2 + i][pl.ds(off, LANES)] * float(N_CLASSES - 1)
        c0 = jnp.minimum(c.astype(jnp.int32), N_CLASSES - 2)
        b = c0 * _WPLANE
        for k in range(_MARK_ROWS):
          o = wcorner[k % 4]
          plane = _WPLANE if k >= 4 else 0
          idx[1 + i][pl.ds(k * ROUND + col, LANES)] = b + o + plane
      return 0

    lax.fori_loop(0, GROUPS, gbody, 0)

  _SUB = 1

  def _seg(m):
    rows = _POS_ROWS if m == 0 else _MARK_ROWS
    return rows * ROUND // _SUB

  def fire(idx, val, sem):
    for m in range(4):
      n = _seg(m)
      for s2 in range(_SUB):
        pltpu.async_copy(map_refs[m].at[idx[m].at[pl.ds(s2 * n, n)]],
                         val[m].at[pl.ds(s2 * n, n)], sem)

  def drain(idx, val, sem):
    for m in range(4):
      n = _seg(m)
      for s2 in range(_SUB):
        pltpu.make_async_copy(map_refs[m].at[idx[m].at[pl.ds(s2 * n, n)]],
                              val[m].at[pl.ds(s2 * n, n)], sem).wait()

  def interp(ci, val):
    def gbody(g, _):
      off = ci * ROUND + g * LANES
      col = g * LANES
      ty = pts_v[0][pl.ds(off, LANES)] * float(H - 1)
      tx = pts_v[1][pl.ds(off, LANES)] * float(W - 1)
      y0 = jnp.minimum(ty.astype(jnp.int32), H - 2)
      x0 = jnp.minimum(tx.astype(jnp.int32), W - 2)
      wy = ty - y0.astype(jnp.float32)
      wx = tx - x0.astype(jnp.float32)
      msk = mask_v[pl.ds(off, LANES)]

      def bilerp(v00, v01, v10, v11):
        top = v00 + wx * (v01 - v00)
        bot = v10 + wx * (v11 - v10)
        return top + wy * (bot - top)

      pvals = [val[0][pl.ds(k * ROUND + col, LANES)] for k in range(_POS_ROWS)]
      out_v[0][pl.ds(off, LANES)] = bilerp(*pvals) * msk
      hi0 = (x0 >> 8) == 1
      hi1 = ((x0 + 1) >> 8) == 1
      for i in range(N_MARKS):
        c = pts_v[2 + i][pl.ds(off, LANES)] * float(N_CLASSES - 1)
        c0 = jnp.minimum(c.astype(jnp.int32), N_CLASSES - 2)
        wc = c - c0.astype(jnp.float32)
        mvals = []
        for k in range(_MARK_ROWS):
          word = val[1 + i][pl.ds(k * ROUND + col, LANES)]
          u = plsc.bitcast(word, jnp.uint32)
          picked = jnp.where(hi1 if (k & 1) else hi0,
                             jnp.bitwise_and(u, jnp.uint32(0xFFFF0000)),
                             u << 16)
          mvals.append(plsc.bitcast(picked, jnp.float32))
        p0 = bilerp(*mvals[0:4])
        p1 = bilerp(*mvals[4:8])
        out_v[1 + i][pl.ds(off, LANES)] = (p0 + wc * (p1 - p0)) * msk
      return 0

    lax.fori_loop(0, GROUPS, gbody, 0)

  compute_idx(0, idx_a)
  fire(idx_a, val_a, sem_a)

  def pair_body(j, _):
    c0 = 2 * j
    compute_idx(c0 + 1, idx_b)
    fire(idx_b, val_b, sem_b)
    drain(idx_a, val_a, sem_a)
    interp(c0, val_a)
    compute_idx(c0 + 2, idx_a)
    fire(idx_a, val_a, sem_a)
    drain(idx_b, val_b, sem_b)
    interp(c0 + 1, val_b)
    return 0

  lax.fori_loop(0, NROUND // 2 - 1, pair_body, 0)

  compute_idx(NROUND - 1, idx_b)
  fire(idx_b, val_b, sem_b)
  drain(idx_a, val_a, sem_a)
  interp(NROUND - 2, val_a)
  drain(idx_b, val_b, sem_b)
  interp(NROUND - 1, val_b)

  for k in range(1 + N_MARKS):
    pltpu.sync_copy(out_v[k], out_hbm.at[pl.ds(k * N_TOTAL + base, N_PER_W)])


@jax.jit
def _sc_call(py, px, c0, c1, c2, mask, pos_map, mm0, mm1, mm2):
  mesh = plsc.VectorSubcoreMesh(core_axis_name="c", subcore_axis_name="s")
  return pl.kernel(
      _sc_body,
      out_type=jax.ShapeDtypeStruct(((1 + N_MARKS) * N_TOTAL,), jnp.float32),
      mesh=mesh,
      compiler_params=pltpu.CompilerParams(needs_layout_passes=False),
      scratch_types=(
          [pltpu.VMEM((N_PER_W,), jnp.float32) for _ in range(5)]
          + [pltpu.VMEM((N_PER_W,), jnp.float32)]
          + [pltpu.VMEM((_POS_ROWS * ROUND,), jnp.int32)]
          + [pltpu.VMEM((_MARK_ROWS * ROUND,), jnp.int32) for _ in range(3)]
          + [pltpu.VMEM((_POS_ROWS * ROUND,), jnp.float32)]
          + [pltpu.VMEM((_MARK_ROWS * ROUND,), jnp.float32) for _ in range(3)]
          + [pltpu.VMEM((_POS_ROWS * ROUND,), jnp.int32)]
          + [pltpu.VMEM((_MARK_ROWS * ROUND,), jnp.int32) for _ in range(3)]
          + [pltpu.VMEM((_POS_ROWS * ROUND,), jnp.float32)]
          + [pltpu.VMEM((_MARK_ROWS * ROUND,), jnp.float32) for _ in range(3)]
          + [pltpu.VMEM((N_PER_W,), jnp.float32) for _ in range(4)]
          + [pltpu.SemaphoreType.DMA, pltpu.SemaphoreType.DMA]
      ),
  )(py, px, c0, c1, c2, mask, pos_map, mm0, mm1, mm2)


def _pack_body(m_ref, out_ref):
  u = jax.lax.bitcast_convert_type(m_ref[0], jnp.uint32)
  rnd = (u + jnp.uint32(0x7FFF) + ((u >> 16) & jnp.uint32(1))) >> 16
  lo = rnd[:, :_WHALF]
  hi = rnd[:, _WHALF:]
  w = jnp.bitwise_or(lo, hi << 16)
  out_ref[...] = jax.lax.bitcast_convert_type(w, jnp.float32).reshape(
      2 * H, 128)


@jax.jit
def _pack_map(m):
  packed = pl.pallas_call(
      _pack_body,
      grid=(N_CLASSES,),
      in_specs=[pl.BlockSpec((1, H, W), lambda c: (c, 0, 0))],
      out_specs=pl.BlockSpec((2 * H, 128), lambda c: (c, 0)),
      out_shape=jax.ShapeDtypeStruct((N_CLASSES * H * 2, 128), jnp.float32),
  )(m)
  return packed.reshape(N_CLASSES * _WPLANE)


def kernel(points, points_mask, position_energy_map,
           marks_energy_map_0, marks_energy_map_1, marks_energy_map_2):
  pts = points.reshape(N_TOTAL, 2 + N_MARKS)
  comps = [pts[:, r] for r in range(5)]
  mask = points_mask.reshape(N_TOTAL)
  pos_map = position_energy_map.reshape(H * W)
  mm0 = _pack_map(marks_energy_map_0)
  mm1 = _pack_map(marks_energy_map_1)
  mm2 = _pack_map(marks_energy_map_2)
  out = _sc_call(*comps, mask, pos_map, mm0, mm1, mm2)
  return out.reshape(1 + N_MARKS, N_SETS, N_POINTS)


# speedup vs baseline: 16.2854x; 1.0815x over previous
import functools

import jax
import jax.numpy as jnp
from jax import lax
from jax.experimental import pallas as pl
from jax.experimental.pallas import tpu as pltpu
from jax.experimental.pallas import tpu_sc as plsc

N_SETS = 16
N_POINTS = 4096
N_MARKS = 3
N_CLASSES = 32
H = W = 512
N_TOTAL = N_SETS * N_POINTS

NC = 2
NS = 16
NW = NC * NS
N_PER_W = N_TOTAL // NW
ROUND = 512
NROUND = N_PER_W // ROUND
LANES = 16
GROUPS = ROUND // LANES

_POS_ROWS = 4
_MARK_ROWS = 8

_WHALF = W // 2
_WPLANE = H * _WHALF


def _common_coords(pts_v, off):
  ty = pts_v[0][pl.ds(off, LANES)] * float(H - 1)
  tx = pts_v[1][pl.ds(off, LANES)] * float(W - 1)
  y0 = jnp.minimum(ty.astype(jnp.int32), H - 2)
  x0 = jnp.minimum(tx.astype(jnp.int32), W - 2)
  return ty, tx, y0, x0


def _make_sc_body(kinds):
  nmaps = len(kinds)
  rows = [_POS_ROWS if k == 'pos' else _MARK_ROWS for k in kinds]

  def body(*refs):
    pts_hbm = refs[0]
    mask_hbm = refs[1]
    map_refs = refs[2:2 + nmaps]
    out_hbm = refs[2 + nmaps]
    scratch = refs[3 + nmaps:]

    pts_v = scratch[0:5]
    mask_v = scratch[5]
    p = 6
    idx_a = scratch[p:p + nmaps]; p += nmaps
    val_a = scratch[p:p + nmaps]; p += nmaps
    idx_b = scratch[p:p + nmaps]; p += nmaps
    val_b = scratch[p:p + nmaps]; p += nmaps
    out_v = scratch[p:p + nmaps]; p += nmaps
    sem_a, sem_b = scratch[p], scratch[p + 1]

    wid = lax.axis_index("s") * NC + lax.axis_index("c")
    base = wid * N_PER_W

    for r in range(5):
      pltpu.sync_copy(pts_hbm[r].at[pl.ds(base, N_PER_W)], pts_v[r])
    pltpu.sync_copy(mask_hbm.at[pl.ds(base, N_PER_W)], mask_v)

    def compute_idx(ci, idx):
      def gbody(g, _):
        off = ci * ROUND + g * LANES
        col = g * LANES
        ty, tx, y0, x0 = _common_coords(pts_v, off)
        xw0 = jnp.bitwise_and(x0, _WHALF - 1)
        xw1 = jnp.bitwise_and(x0 + 1, _WHALF - 1)
        row0 = y0 * _WHALF
        wcorner = (row0 + xw0, row0 + xw1,
                   row0 + _WHALF + xw0, row0 + _WHALF + xw1)
        s00 = y0 * W + x0
        for m, kind in enumerate(kinds):
          if kind == 'pos':
            for k, o in enumerate((0, 1, W, W + 1)):
              idx[m][pl.ds(k * ROUND + col, LANES)] = s00 + o
          else:
            c = pts_v[2 + kind][pl.ds(off, LANES)] * float(N_CLASSES - 1)
            c0 = jnp.minimum(c.astype(jnp.int32), N_CLASSES - 2)
            b = c0 * _WPLANE
            for k in range(_MARK_ROWS):
              plane = _WPLANE if k >= 4 else 0
              idx[m][pl.ds(k * ROUND + col, LANES)] = (
                  b + wcorner[k % 4] + plane)
        return 0

      lax.fori_loop(0, GROUPS, gbody, 0)

    def fire(idx, val, sem):
      for m in range(nmaps):
        pltpu.async_copy(map_refs[m].at[idx[m]], val[m], sem)

    def drain(idx, val, sem):
      for m in range(nmaps):
        pltpu.make_async_copy(map_refs[m].at[idx[m]], val[m], sem).wait()

    def interp(ci, val):
      def gbody(g, _):
        off = ci * ROUND + g * LANES
        col = g * LANES
        ty, tx, y0, x0 = _common_coords(pts_v, off)
        wy = ty - y0.astype(jnp.float32)
        wx = tx - x0.astype(jnp.float32)
        msk = mask_v[pl.ds(off, LANES)]
        hi0 = (x0 >> 8) == 1
        hi1 = ((x0 + 1) >> 8) == 1

        def bilerp(v00, v01, v10, v11):
          top = v00 + wx * (v01 - v00)
          bot = v10 + wx * (v11 - v10)
          return top + wy * (bot - top)

        for m, kind in enumerate(kinds):
          if kind == 'pos':
            pvals = [val[m][pl.ds(k * ROUND + col, LANES)]
                     for k in range(_POS_ROWS)]
            out_v[m][pl.ds(off, LANES)] = bilerp(*pvals) * msk
          else:
            c = pts_v[2 + kind][pl.ds(off, LANES)] * float(N_CLASSES - 1)
            c0 = jnp.minimum(c.astype(jnp.int32), N_CLASSES - 2)
            wc = c - c0.astype(jnp.float32)
            mvals = []
            for k in range(_MARK_ROWS):
              word = val[m][pl.ds(k * ROUND + col, LANES)]
              u = plsc.bitcast(word, jnp.uint32)
              picked = jnp.where(hi1 if (k & 1) else hi0,
                                 jnp.bitwise_and(u, jnp.uint32(0xFFFF0000)),
                                 u << 16)
              mvals.append(plsc.bitcast(picked, jnp.float32))
            p0 = bilerp(*mvals[0:4])
            p1 = bilerp(*mvals[4:8])
            out_v[m][pl.ds(off, LANES)] = (p0 + wc * (p1 - p0)) * msk
        return 0

      lax.fori_loop(0, GROUPS, gbody, 0)

    compute_idx(0, idx_a)
    fire(idx_a, val_a, sem_a)

    def pair_body(j, _):
      c0 = 2 * j
      compute_idx(c0 + 1, idx_b)
      fire(idx_b, val_b, sem_b)
      drain(idx_a, val_a, sem_a)
      interp(c0, val_a)
      compute_idx(c0 + 2, idx_a)
      fire(idx_a, val_a, sem_a)
      drain(idx_b, val_b, sem_b)
      interp(c0 + 1, val_b)
      return 0

    lax.fori_loop(0, NROUND // 2 - 1, pair_body, 0)

    compute_idx(NROUND - 1, idx_b)
    fire(idx_b, val_b, sem_b)
    drain(idx_a, val_a, sem_a)
    interp(NROUND - 2, val_a)
    drain(idx_b, val_b, sem_b)
    interp(NROUND - 1, val_b)

    for m in range(nmaps):
      pltpu.sync_copy(out_v[m], out_hbm.at[pl.ds(m * N_TOTAL + base, N_PER_W)])

  return body, rows


def _make_sc_call(kinds):
  body, rows = _make_sc_body(kinds)
  nmaps = len(kinds)
  mesh = plsc.VectorSubcoreMesh(core_axis_name="c", subcore_axis_name="s")

  def call(comps, mask, maps):
    return pl.kernel(
        body,
        out_type=jax.ShapeDtypeStruct((nmaps * N_TOTAL,), jnp.float32),
        mesh=mesh,
        compiler_params=pltpu.CompilerParams(needs_layout_passes=False),
        scratch_types=(
            [pltpu.VMEM((N_PER_W,), jnp.float32) for _ in range(5)]
            + [pltpu.VMEM((N_PER_W,), jnp.float32)]
            + [pltpu.VMEM((r * ROUND,), jnp.int32) for r in rows]
            + [pltpu.VMEM((r * ROUND,), jnp.float32) for r in rows]
            + [pltpu.VMEM((r * ROUND,), jnp.int32) for r in rows]
            + [pltpu.VMEM((r * ROUND,), jnp.float32) for r in rows]
            + [pltpu.VMEM((N_PER_W,), jnp.float32) for _ in range(nmaps)]
            + [pltpu.SemaphoreType.DMA, pltpu.SemaphoreType.DMA]
        ),
    )(comps, mask, *maps)

  return call


_sc_call_a = _make_sc_call(('pos', 0))
_sc_call_b = _make_sc_call((1, 2))


def _pack_body(m_ref, out_ref):
  u = jax.lax.bitcast_convert_type(m_ref[0], jnp.uint32)
  rnd = (u + jnp.uint32(0x7FFF) + ((u >> 16) & jnp.uint32(1))) >> 16
  lo = rnd[:, :_WHALF]
  hi = rnd[:, _WHALF:]
  w = jnp.bitwise_or(lo, hi << 16)
  out_ref[...] = jax.lax.bitcast_convert_type(w, jnp.float32).reshape(
      2 * H, 128)


def _pack_map(m):
  packed = pl.pallas_call(
      _pack_body,
      grid=(N_CLASSES,),
      in_specs=[pl.BlockSpec((1, H, W), lambda c: (c, 0, 0))],
      out_specs=pl.BlockSpec((2 * H, 128), lambda c: (c, 0)),
      out_shape=jax.ShapeDtypeStruct((N_CLASSES * H * 2, 128), jnp.float32),
  )(m)
  return packed.reshape(N_CLASSES * _WPLANE)


@jax.jit
def _run(points, points_mask, position_energy_map,
         marks_energy_map_0, marks_energy_map_1, marks_energy_map_2):
  pts = points.reshape(N_TOTAL, 2 + N_MARKS)
  comps = [pts[:, r] for r in range(5)]
  mask = points_mask.reshape(N_TOTAL)
  pos_map = position_energy_map.reshape(H * W)
  mm0 = _pack_map(marks_energy_map_0)
  mm1 = _pack_map(marks_energy_map_1)
  mm2 = _pack_map(marks_energy_map_2)
  out_a = _sc_call_a(comps, mask, (pos_map, mm0))
  out_b = _sc_call_b(comps, mask, (mm1, mm2))
  out = jnp.concatenate(
      [out_a.reshape(2, N_TOTAL), out_b.reshape(2, N_TOTAL)], axis=0)
  return out.reshape(1 + N_MARKS, N_SETS, N_POINTS)


def kernel(points, points_mask, position_energy_map,
           marks_energy_map_0, marks_energy_map_1, marks_energy_map_2):
  return _run(points, points_mask, position_energy_map,
              marks_energy_map_0, marks_energy_map_1, marks_energy_map_2)
